# Initial kernel scaffold; baseline (speedup 1.0000x reference)
import functools

import jax
import jax.numpy as jnp
from jax import lax
from jax.experimental import pallas as pl
from jax.experimental.pallas import tpu as pltpu
from jax.experimental.pallas import tpu_sc as plsc

N = 10000
NPAD = 10240
NC, NS = 2, 16
RPT = NPAD // NS
CHUNK = 128
D_IN, D_HID, D_OUT = 256, 256, 128
H1 = D_HID // NC
H2 = D_OUT // NC
DEGW = 16
RB = 1000

_mesh = functools.partial(
    plsc.VectorSubcoreMesh, core_axis_name="c", subcore_axis_name="s",
    num_cores=NC, num_subcores=NS)


def _deg_body(nchunks, dst_hbm, ones_hbm, zeros_hbm, dega, degb,
              acc, idx, ones_v):
  c = lax.axis_index("c")
  s = lax.axis_index("s")
  pltpu.sync_copy(zeros_hbm, acc.at[pl.ds(s * RPT, RPT)])
  pltpu.sync_copy(ones_hbm, ones_v)
  plsc.subcore_barrier()
  base = (c * NS + s) * nchunks * CHUNK

  def step(k, carry):
    pltpu.sync_copy(dst_hbm.at[pl.ds(base + k * CHUNK, CHUNK)], idx.at[0])
    pltpu.sync_copy(ones_v, acc.at[idx.at[0]], add=True)
    return carry

  lax.fori_loop(0, nchunks, step, 0)
  plsc.subcore_barrier()

  @pl.when(c == 0)
  def _():
    pltpu.sync_copy(acc.at[pl.ds(s * RPT, RPT)], dega.at[pl.ds(s * RPT, RPT)])

  @pl.when(c == 1)
  def _():
    pltpu.sync_copy(acc.at[pl.ds(s * RPT, RPT)], degb.at[pl.ds(s * RPT, RPT)])


def _deg_kernel(dst_pad, ones, zeros16):
  nchunks = dst_pad.shape[0] // (NC * NS * CHUNK)
  return pl.kernel(
      functools.partial(_deg_body, nchunks),
      out_type=(jax.ShapeDtypeStruct((NPAD, DEGW), jnp.float32),
                jax.ShapeDtypeStruct((NPAD, DEGW), jnp.float32)),
      mesh=_mesh(),
      scratch_types=[
          pltpu.VMEM_SHARED((NPAD, DEGW), jnp.float32),
          pltpu.VMEM((1, CHUNK), jnp.int32),
          pltpu.VMEM((CHUNK, DEGW), jnp.float32),
      ],
  )(dst_pad, ones, zeros16)


def _agg_body(nchunks, hw, ga_hbm, gb_hbm, src_hbm, dst_hbm, zeros_hbm,
              outa, outb, acc, idxs, idxd, rows):
  c = lax.axis_index("c")
  s = lax.axis_index("s")
  pltpu.sync_copy(zeros_hbm, acc.at[pl.ds(s * RPT, RPT)])
  plsc.subcore_barrier()
  base = s * nchunks * CHUNK

  def step(k, carry):
    off = base + k * CHUNK
    pltpu.sync_copy(src_hbm.at[pl.ds(off, CHUNK)], idxs.at[0])
    pltpu.sync_copy(dst_hbm.at[pl.ds(off, CHUNK)], idxd.at[0])

    @pl.when(c == 0)
    def _():
      pltpu.sync_copy(ga_hbm.at[idxs.at[0]], rows)

    @pl.when(c == 1)
    def _():
      pltpu.sync_copy(gb_hbm.at[idxs.at[0]], rows)

    pltpu.sync_copy(rows, acc.at[idxd.at[0]], add=True)
    return carry

  lax.fori_loop(0, nchunks, step, 0)
  plsc.subcore_barrier()

  @pl.when(c == 0)
  def _():
    pltpu.sync_copy(acc.at[pl.ds(s * RPT, RPT)], outa.at[pl.ds(s * RPT, RPT)])

  @pl.when(c == 1)
  def _():
    pltpu.sync_copy(acc.at[pl.ds(s * RPT, RPT)], outb.at[pl.ds(s * RPT, RPT)])


def _agg_kernel(ga, gb, src_pad, dst_pad, zeros, hw):
  nchunks = src_pad.shape[0] // (NS * CHUNK)
  return pl.kernel(
      functools.partial(_agg_body, nchunks, hw),
      out_type=(jax.ShapeDtypeStruct((NPAD, hw), jnp.float32),
                jax.ShapeDtypeStruct((NPAD, hw), jnp.float32)),
      mesh=_mesh(),
      scratch_types=[
          pltpu.VMEM_SHARED((NPAD, hw), jnp.float32),
          pltpu.VMEM((1, CHUNK), jnp.int32),
          pltpu.VMEM((1, CHUNK), jnp.int32),
          pltpu.VMEM((CHUNK, hw), jnp.float32),
      ],
  )(ga, gb, src_pad, dst_pad, zeros)


def _dis(dega, degb):
  deg = dega[:, 0] + degb[:, 0] + 1.0
  return lax.rsqrt(deg)


def _mm1_body(x_ref, w_ref, dega_ref, degb_ref, g1a_ref, g1b_ref):
  h = jnp.dot(x_ref[...], w_ref[...], preferred_element_type=jnp.float32)
  g = h * _dis(dega_ref[...], degb_ref[...])[:, None]
  g1a_ref[...] = g[:, :H1]
  g1b_ref[...] = g[:, H1:]


def _mm1(x, w1, dega, degb):
  return pl.pallas_call(
      _mm1_body,
      grid=(N // RB,),
      in_specs=[
          pl.BlockSpec((RB, D_IN), lambda i: (i, 0)),
          pl.BlockSpec((D_IN, D_HID), lambda i: (0, 0)),
          pl.BlockSpec((RB, DEGW), lambda i: (i, 0)),
          pl.BlockSpec((RB, DEGW), lambda i: (i, 0)),
      ],
      out_specs=(pl.BlockSpec((RB, H1), lambda i: (i, 0)),
                 pl.BlockSpec((RB, H1), lambda i: (i, 0))),
      out_shape=(jax.ShapeDtypeStruct((N, H1), jnp.float32),
                 jax.ShapeDtypeStruct((N, H1), jnp.float32)),
  )(x, w1, dega, degb)


def _mm2_body(a1a_ref, a1b_ref, g1a_ref, g1b_ref, dega_ref, degb_ref,
              w2_ref, b1_ref, g2a_ref, g2b_ref):
  dis = _dis(dega_ref[...], degb_ref[...])[:, None]
  z0 = jnp.maximum(dis * (a1a_ref[...] + g1a_ref[...]) + b1_ref[0, :H1], 0.0)
  z1 = jnp.maximum(dis * (a1b_ref[...] + g1b_ref[...]) + b1_ref[0, H1:], 0.0)
  z = jnp.concatenate([z0, z1], axis=1)
  g2 = jnp.dot(z, w2_ref[...], preferred_element_type=jnp.float32) * dis
  g2a_ref[...] = g2[:, :H2]
  g2b_ref[...] = g2[:, H2:]


def _mm2(a1a, a1b, g1a, g1b, dega, degb, w2, b1):
  return pl.pallas_call(
      _mm2_body,
      grid=(N // RB,),
      in_specs=[
          pl.BlockSpec((RB, H1), lambda i: (i, 0)),
          pl.BlockSpec((RB, H1), lambda i: (i, 0)),
          pl.BlockSpec((RB, H1), lambda i: (i, 0)),
          pl.BlockSpec((RB, H1), lambda i: (i, 0)),
          pl.BlockSpec((RB, DEGW), lambda i: (i, 0)),
          pl.BlockSpec((RB, DEGW), lambda i: (i, 0)),
          pl.BlockSpec((D_HID, D_OUT), lambda i: (0, 0)),
          pl.BlockSpec((1, D_HID), lambda i: (0, 0)),
      ],
      out_specs=(pl.BlockSpec((RB, H2), lambda i: (i, 0)),
                 pl.BlockSpec((RB, H2), lambda i: (i, 0))),
      out_shape=(jax.ShapeDtypeStruct((N, H2), jnp.float32),
                 jax.ShapeDtypeStruct((N, H2), jnp.float32)),
  )(a1a, a1b, g1a, g1b, dega, degb, w2, b1)


def _fin_body(a2a_ref, a2b_ref, g2a_ref, g2b_ref, dega_ref, degb_ref,
              b2_ref, out_ref):
  dis = _dis(dega_ref[...], degb_ref[...])[:, None]
  o0 = dis * (a2a_ref[...] + g2a_ref[...]) + b2_ref[0, :H2]
  o1 = dis * (a2b_ref[...] + g2b_ref[...]) + b2_ref[0, H2:]
  out_ref[...] = jnp.concatenate([o0, o1], axis=1)


def _fin(a2a, a2b, g2a, g2b, dega, degb, b2):
  return pl.pallas_call(
      _fin_body,
      grid=(N // RB,),
      in_specs=[
          pl.BlockSpec((RB, H2), lambda i: (i, 0)),
          pl.BlockSpec((RB, H2), lambda i: (i, 0)),
          pl.BlockSpec((RB, H2), lambda i: (i, 0)),
          pl.BlockSpec((RB, H2), lambda i: (i, 0)),
          pl.BlockSpec((RB, DEGW), lambda i: (i, 0)),
          pl.BlockSpec((RB, DEGW), lambda i: (i, 0)),
          pl.BlockSpec((1, D_OUT), lambda i: (0, 0)),
      ],
      out_specs=pl.BlockSpec((RB, D_OUT), lambda i: (i, 0)),
      out_shape=jax.ShapeDtypeStruct((N, D_OUT), jnp.float32),
  )(a2a, a2b, g2a, g2b, dega, degb, b2)


def kernel(x, edge_index, W1, b1, W2, b2):
  e = edge_index.shape[1]
  gran = NC * NS * CHUNK
  epad = ((e + gran - 1) // gran) * gran
  src = jnp.concatenate(
      [edge_index[0], jnp.zeros((epad - e,), jnp.int32)])
  dst = jnp.concatenate(
      [edge_index[1], jnp.full((epad - e,), N, jnp.int32)])

  ones = jnp.ones((CHUNK, DEGW), jnp.float32)
  zeros16 = jnp.zeros((RPT, DEGW), jnp.float32)
  zeros_h1 = jnp.zeros((RPT, H1), jnp.float32)
  zeros_h2 = jnp.zeros((RPT, H2), jnp.float32)

  dega, degb = _deg_kernel(dst, ones, zeros16)
  dega, degb = dega[:N], degb[:N]
  g1a, g1b = _mm1(x, W1, dega, degb)
  a1a, a1b = _agg_kernel(g1a, g1b, src, dst, zeros_h1, H1)
  g2a, g2b = _mm2(a1a[:N], a1b[:N], g1a, g1b, dega, degb, W2,
                  b1.reshape(1, D_HID))
  a2a, a2b = _agg_kernel(g2a, g2b, src, dst, zeros_h2, H2)
  return _fin(a2a[:N], a2b[:N], g2a, g2b, dega, degb,
              b2.reshape(1, D_OUT))


# speedup vs baseline: 6.3818x; 6.3818x over previous
import functools

import jax
import jax.numpy as jnp
from jax import lax
from jax.experimental import pallas as pl
from jax.experimental.pallas import tpu as pltpu
from jax.experimental.pallas import tpu_sc as plsc

N = 10000
NPAD = 10240
NC, NS = 2, 16
RPT = NPAD // NS
CHUNK = 128
D_IN, D_HID, D_OUT = 256, 256, 128
H1 = D_HID // NC
W128 = 128
DEGW = 16
RB = 1000

_mesh = functools.partial(
    plsc.VectorSubcoreMesh, core_axis_name="c", subcore_axis_name="s",
    num_cores=NC, num_subcores=NS)


def _deg_body(nchunks, dst_hbm, ones_hbm, zeros_hbm, deg_out,
              acc, idx, ones_v):
  c = lax.axis_index("c")
  s = lax.axis_index("s")
  pltpu.sync_copy(zeros_hbm, acc.at[pl.ds(s * RPT, RPT)])
  pltpu.sync_copy(ones_hbm, ones_v)
  plsc.subcore_barrier()
  base = (c * NS + s) * nchunks

  def step(k, carry):
    pltpu.sync_copy(dst_hbm.at[pl.ds((base + k) * CHUNK, CHUNK)], idx)
    pltpu.sync_copy(ones_v, acc.at[idx], add=True)
    return carry

  lax.fori_loop(0, nchunks, step, 0)
  plsc.subcore_barrier()
  pltpu.sync_copy(acc.at[pl.ds(s * RPT, RPT)],
                  deg_out.at[c, pl.ds(s * RPT, RPT)])


def _deg_kernel(dst_pad, ones, zeros):
  nchunks = dst_pad.shape[0] // (NC * NS * CHUNK)
  return pl.kernel(
      functools.partial(_deg_body, nchunks),
      out_type=jax.ShapeDtypeStruct((NC, NPAD, W128), jnp.float32),
      mesh=_mesh(),
      scratch_types=[
          pltpu.VMEM_SHARED((NPAD, W128), jnp.float32),
          pltpu.VMEM((CHUNK,), jnp.int32),
          pltpu.VMEM((CHUNK, W128), jnp.float32),
      ],
  )(dst_pad, ones, zeros)


def _agg_body(nchunks, epad, gcat_hbm, srcq_hbm, dst_hbm, zeros_hbm,
              out, acc, idxs, idxd, rows):
  c = lax.axis_index("c")
  s = lax.axis_index("s")
  pltpu.sync_copy(zeros_hbm, acc.at[pl.ds(s * RPT, RPT)])
  plsc.subcore_barrier()
  base = s * nchunks

  def step(k, carry):
    r = base + k
    pltpu.sync_copy(srcq_hbm.at[pl.ds(c * epad + r * CHUNK, CHUNK)], idxs)
    pltpu.sync_copy(dst_hbm.at[pl.ds(r * CHUNK, CHUNK)], idxd)
    pltpu.sync_copy(gcat_hbm.at[idxs], rows)
    pltpu.sync_copy(rows, acc.at[idxd], add=True)
    return carry

  lax.fori_loop(0, nchunks, step, 0)
  plsc.subcore_barrier()
  pltpu.sync_copy(acc.at[pl.ds(s * RPT, RPT)],
                  out.at[c, pl.ds(s * RPT, RPT)])


def _agg_kernel(gcat, srcq, dst_pad, zeros):
  epad = dst_pad.shape[0]
  nchunks = epad // (NS * CHUNK)
  return pl.kernel(
      functools.partial(_agg_body, nchunks, epad),
      out_type=jax.ShapeDtypeStruct((NC, NPAD, H1), jnp.float32),
      mesh=_mesh(),
      scratch_types=[
          pltpu.VMEM_SHARED((NPAD, H1), jnp.float32),
          pltpu.VMEM((CHUNK,), jnp.int32),
          pltpu.VMEM((CHUNK,), jnp.int32),
          pltpu.VMEM((CHUNK, H1), jnp.float32),
      ],
  )(gcat, srcq, dst_pad, zeros)


def _agg2_body(nchunks, g_hbm, src_hbm, dst_hbm, zeros_hbm,
               out, acc, idxs, idxd, rows):
  c = lax.axis_index("c")
  s = lax.axis_index("s")
  pltpu.sync_copy(zeros_hbm, acc.at[pl.ds(s * RPT, RPT)])
  plsc.subcore_barrier()
  base = (c * NS + s) * nchunks

  def step(k, carry):
    r = base + k
    pltpu.sync_copy(src_hbm.at[pl.ds(r * CHUNK, CHUNK)], idxs)
    pltpu.sync_copy(dst_hbm.at[pl.ds(r * CHUNK, CHUNK)], idxd)
    pltpu.sync_copy(g_hbm.at[idxs], rows)
    pltpu.sync_copy(rows, acc.at[idxd], add=True)
    return carry

  lax.fori_loop(0, nchunks, step, 0)
  plsc.subcore_barrier()
  pltpu.sync_copy(acc.at[pl.ds(s * RPT, RPT)],
                  out.at[c, pl.ds(s * RPT, RPT)])


def _agg2_kernel(g, src_pad, dst_pad, zeros):
  nchunks = src_pad.shape[0] // (NC * NS * CHUNK)
  return pl.kernel(
      functools.partial(_agg2_body, nchunks),
      out_type=jax.ShapeDtypeStruct((NC, NPAD, D_OUT), jnp.float32),
      mesh=_mesh(),
      scratch_types=[
          pltpu.VMEM_SHARED((NPAD, D_OUT), jnp.float32),
          pltpu.VMEM((CHUNK,), jnp.int32),
          pltpu.VMEM((CHUNK,), jnp.int32),
          pltpu.VMEM((CHUNK, D_OUT), jnp.float32),
      ],
  )(g, src_pad, dst_pad, zeros)


def _dis(dega, degb):
  deg = dega[:, 0] + degb[:, 0] + 1.0
  return lax.rsqrt(deg)


def _mm1_body(x_ref, w_ref, dega_ref, degb_ref, gc_ref):
  h = jnp.dot(x_ref[...], w_ref[...], preferred_element_type=jnp.float32)
  g = h * _dis(dega_ref[...], degb_ref[...])[:, None]
  gc_ref[0] = g[:, :H1]
  gc_ref[1] = g[:, H1:]


def _mm1(x, w1, dega, degb):
  return pl.pallas_call(
      _mm1_body,
      grid=(N // RB,),
      in_specs=[
          pl.BlockSpec((RB, D_IN), lambda i: (i, 0)),
          pl.BlockSpec((D_IN, D_HID), lambda i: (0, 0)),
          pl.BlockSpec((RB, DEGW), lambda i: (i, 0)),
          pl.BlockSpec((RB, DEGW), lambda i: (i, 0)),
      ],
      out_specs=pl.BlockSpec((2, RB, H1), lambda i: (0, i, 0)),
      out_shape=jax.ShapeDtypeStruct((2, N, H1), jnp.float32),
  )(x, w1, dega, degb)


def _mm2_body(a1a_ref, a1b_ref, gc_ref, dega_ref, degb_ref,
              w2_ref, b1_ref, g2_ref):
  dis = _dis(dega_ref[...], degb_ref[...])[:, None]
  z0 = jnp.maximum(dis * (a1a_ref[...] + gc_ref[0]) + b1_ref[0, :H1], 0.0)
  z1 = jnp.maximum(dis * (a1b_ref[...] + gc_ref[1]) + b1_ref[0, H1:], 0.0)
  z = jnp.concatenate([z0, z1], axis=1)
  g2_ref[...] = jnp.dot(z, w2_ref[...], preferred_element_type=jnp.float32) * dis


def _mm2(a1a, a1b, gc, dega, degb, w2, b1):
  return pl.pallas_call(
      _mm2_body,
      grid=(N // RB,),
      in_specs=[
          pl.BlockSpec((RB, H1), lambda i: (i, 0)),
          pl.BlockSpec((RB, H1), lambda i: (i, 0)),
          pl.BlockSpec((2, RB, H1), lambda i: (0, i, 0)),
          pl.BlockSpec((RB, DEGW), lambda i: (i, 0)),
          pl.BlockSpec((RB, DEGW), lambda i: (i, 0)),
          pl.BlockSpec((D_HID, D_OUT), lambda i: (0, 0)),
          pl.BlockSpec((1, D_HID), lambda i: (0, 0)),
      ],
      out_specs=pl.BlockSpec((RB, D_OUT), lambda i: (i, 0)),
      out_shape=jax.ShapeDtypeStruct((N, D_OUT), jnp.float32),
  )(a1a, a1b, gc, dega, degb, w2, b1)


def _fin_body(a2a_ref, a2b_ref, g2_ref, dega_ref, degb_ref,
              b2_ref, out_ref):
  dis = _dis(dega_ref[...], degb_ref[...])[:, None]
  out_ref[...] = (dis * (a2a_ref[...] + a2b_ref[...] + g2_ref[...])
                  + b2_ref[0, :])


def _fin(a2a, a2b, g2, dega, degb, b2):
  return pl.pallas_call(
      _fin_body,
      grid=(N // RB,),
      in_specs=[
          pl.BlockSpec((RB, D_OUT), lambda i: (i, 0)),
          pl.BlockSpec((RB, D_OUT), lambda i: (i, 0)),
          pl.BlockSpec((RB, D_OUT), lambda i: (i, 0)),
          pl.BlockSpec((RB, DEGW), lambda i: (i, 0)),
          pl.BlockSpec((RB, DEGW), lambda i: (i, 0)),
          pl.BlockSpec((1, D_OUT), lambda i: (0, 0)),
      ],
      out_specs=pl.BlockSpec((RB, D_OUT), lambda i: (i, 0)),
      out_shape=jax.ShapeDtypeStruct((N, D_OUT), jnp.float32),
  )(a2a, a2b, g2, dega, degb, b2)


def kernel(x, edge_index, W1, b1, W2, b2):
  e = edge_index.shape[1]
  gran = NC * NS * CHUNK
  epad = ((e + gran - 1) // gran) * gran
  src = jnp.concatenate(
      [edge_index[0], jnp.zeros((epad - e,), jnp.int32)])
  dst = jnp.concatenate(
      [edge_index[1], jnp.full((epad - e,), N, jnp.int32)])
  srcq = jnp.concatenate([src, src + N])

  ones = jnp.ones((CHUNK, W128), jnp.float32)
  zeros128 = jnp.zeros((RPT, W128), jnp.float32)
  zeros_h1 = jnp.zeros((RPT, H1), jnp.float32)
  zeros_h2 = jnp.zeros((RPT, D_OUT), jnp.float32)

  deg2 = _deg_kernel(dst, ones, zeros128)
  dega, degb = deg2[0, :N, :DEGW], deg2[1, :N, :DEGW]
  gc = _mm1(x, W1, dega, degb)
  gcat = gc.reshape(2 * N, H1)
  a1 = _agg_kernel(gcat, srcq, dst, zeros_h1)
  g2 = _mm2(a1[0, :N], a1[1, :N], gc, dega, degb, W2, b1.reshape(1, D_HID))
  a2 = _agg2_kernel(g2, src, dst, zeros_h2)
  return _fin(a2[0, :N], a2[1, :N], g2, dega, degb, b2.reshape(1, D_OUT))


# speedup vs baseline: 7.6049x; 1.1916x over previous
import functools

import jax
import jax.numpy as jnp
from jax import lax
from jax.experimental import pallas as pl
from jax.experimental.pallas import tpu as pltpu
from jax.experimental.pallas import tpu_sc as plsc

N = 10000
NPAD = 10240
NC, NS = 2, 16
RPT = NPAD // NS
CHUNK = 128
GRP = 4
AGRP = 2
D_IN, D_HID, D_OUT = 256, 256, 128
H1 = D_HID // NC
W128 = 128
DEGW = 16
RB = 1000

_mesh = functools.partial(
    plsc.VectorSubcoreMesh, core_axis_name="c", subcore_axis_name="s",
    num_cores=NC, num_subcores=NS)


def _deg_body(ngrp, dst_hbm, ones_hbm, zeros_hbm, deg_out,
              acc, idx0, idx1, idx2, idx3, ones_v, isem, ssem):
  c = lax.axis_index("c")
  s = lax.axis_index("s")
  idxd = [idx0, idx1, idx2, idx3]
  pltpu.sync_copy(zeros_hbm, acc.at[pl.ds(s * RPT, RPT)])
  pltpu.sync_copy(ones_hbm, ones_v)
  plsc.subcore_barrier()
  base = (c * NS + s) * ngrp * GRP

  def step(k, carry):
    q = (base + k * GRP) * CHUNK
    ld = [pltpu.async_copy(dst_hbm.at[pl.ds(q + j * CHUNK, CHUNK)],
                           idxd[j], isem) for j in range(GRP)]
    st = []
    for j in range(GRP):
      ld[j].wait()
      st.append(pltpu.async_copy(ones_v, acc.at[idxd[j]], ssem, add=True))
    for d in st:
      d.wait()
    return carry

  lax.fori_loop(0, ngrp, step, 0)
  plsc.subcore_barrier()
  pltpu.sync_copy(acc.at[pl.ds(s * RPT, RPT)],
                  deg_out.at[c, pl.ds(s * RPT, RPT)])


def _deg_kernel(dst_pad, ones, zeros):
  ngrp = dst_pad.shape[0] // (NC * NS * CHUNK * GRP)
  return pl.kernel(
      functools.partial(_deg_body, ngrp),
      out_type=jax.ShapeDtypeStruct((NC, NPAD, W128), jnp.float32),
      mesh=_mesh(),
      scratch_types=[
          pltpu.VMEM_SHARED((NPAD, W128), jnp.float32),
          pltpu.VMEM((CHUNK,), jnp.int32),
          pltpu.VMEM((CHUNK,), jnp.int32),
          pltpu.VMEM((CHUNK,), jnp.int32),
          pltpu.VMEM((CHUNK,), jnp.int32),
          pltpu.VMEM((CHUNK, W128), jnp.float32),
          pltpu.SemaphoreType.DMA,
          pltpu.SemaphoreType.DMA,
      ],
  )(dst_pad, ones, zeros)


def _agg_pipeline(ngrp, base, src_off, g_hbm, src_hbm, dst_hbm,
                  acc, idxs, idxd, rows, isem, gsem, ssem):
  grp = len(rows)

  def step(k, carry):
    q = (base + k * grp) * CHUNK
    ld = []
    for j in range(grp):
      ld.append(pltpu.async_copy(
          src_hbm.at[pl.ds(src_off + q + j * CHUNK, CHUNK)], idxs[j], isem))
      ld.append(pltpu.async_copy(
          dst_hbm.at[pl.ds(q + j * CHUNK, CHUNK)], idxd[j], isem))
    gt = []
    for j in range(grp):
      ld[2 * j].wait()
      ld[2 * j + 1].wait()
      gt.append(pltpu.async_copy(g_hbm.at[idxs[j]], rows[j], gsem))
    st = []
    for j in range(grp):
      gt[j].wait()
      st.append(pltpu.async_copy(rows[j], acc.at[idxd[j]], ssem, add=True))
    for d in st:
      d.wait()
    return carry

  lax.fori_loop(0, ngrp, step, 0)


def _agg_body(ngrp, epad, gcat_hbm, srcq_hbm, dst_hbm, zeros_hbm,
              out, acc, is0, is1, id0, id1, r0, r1, isem, gsem, ssem):
  c = lax.axis_index("c")
  s = lax.axis_index("s")
  pltpu.sync_copy(zeros_hbm, acc.at[pl.ds(s * RPT, RPT)])
  plsc.subcore_barrier()
  _agg_pipeline(ngrp, s * ngrp * AGRP, c * epad, gcat_hbm, srcq_hbm, dst_hbm,
                acc, [is0, is1], [id0, id1], [r0, r1], isem, gsem, ssem)
  plsc.subcore_barrier()
  pltpu.sync_copy(acc.at[pl.ds(s * RPT, RPT)],
                  out.at[c, pl.ds(s * RPT, RPT)])


def _agg_scratch(hw):
  return ([pltpu.VMEM_SHARED((NPAD, hw), jnp.float32)]
          + [pltpu.VMEM((CHUNK,), jnp.int32) for _ in range(2 * AGRP)]
          + [pltpu.VMEM((CHUNK, hw), jnp.float32) for _ in range(AGRP)]
          + [pltpu.SemaphoreType.DMA] * 3)


def _agg_kernel(gcat, srcq, dst_pad, zeros):
  epad = dst_pad.shape[0]
  ngrp = epad // (NS * CHUNK * AGRP)
  return pl.kernel(
      functools.partial(_agg_body, ngrp, epad),
      out_type=jax.ShapeDtypeStruct((NC, NPAD, H1), jnp.float32),
      mesh=_mesh(),
      scratch_types=_agg_scratch(H1),
  )(gcat, srcq, dst_pad, zeros)


def _agg2_body(ngrp, g_hbm, src_hbm, dst_hbm, zeros_hbm,
               out, acc, is0, is1, id0, id1, r0, r1, isem, gsem, ssem):
  c = lax.axis_index("c")
  s = lax.axis_index("s")
  pltpu.sync_copy(zeros_hbm, acc.at[pl.ds(s * RPT, RPT)])
  plsc.subcore_barrier()
  _agg_pipeline(ngrp, (c * NS + s) * ngrp * AGRP, 0, g_hbm, src_hbm, dst_hbm,
                acc, [is0, is1], [id0, id1], [r0, r1], isem, gsem, ssem)
  plsc.subcore_barrier()
  pltpu.sync_copy(acc.at[pl.ds(s * RPT, RPT)],
                  out.at[c, pl.ds(s * RPT, RPT)])


def _agg2_kernel(g, src_pad, dst_pad, zeros):
  ngrp = src_pad.shape[0] // (NC * NS * CHUNK * AGRP)
  return pl.kernel(
      functools.partial(_agg2_body, ngrp),
      out_type=jax.ShapeDtypeStruct((NC, NPAD, D_OUT), jnp.float32),
      mesh=_mesh(),
      scratch_types=_agg_scratch(D_OUT),
  )(g, src_pad, dst_pad, zeros)


def _dis(dega, degb):
  deg = dega[:, 0] + degb[:, 0] + 1.0
  return lax.rsqrt(deg)


def _mm1_body(x_ref, w_ref, dega_ref, degb_ref, gc_ref):
  h = jnp.dot(x_ref[...], w_ref[...], preferred_element_type=jnp.float32)
  g = h * _dis(dega_ref[...], degb_ref[...])[:, None]
  gc_ref[0] = g[:, :H1]
  gc_ref[1] = g[:, H1:]


def _mm1(x, w1, dega, degb):
  return pl.pallas_call(
      _mm1_body,
      grid=(N // RB,),
      in_specs=[
          pl.BlockSpec((RB, D_IN), lambda i: (i, 0)),
          pl.BlockSpec((D_IN, D_HID), lambda i: (0, 0)),
          pl.BlockSpec((RB, DEGW), lambda i: (i, 0)),
          pl.BlockSpec((RB, DEGW), lambda i: (i, 0)),
      ],
      out_specs=pl.BlockSpec((2, RB, H1), lambda i: (0, i, 0)),
      out_shape=jax.ShapeDtypeStruct((2, N, H1), jnp.float32),
  )(x, w1, dega, degb)


def _mm2_body(a1a_ref, a1b_ref, gc_ref, dega_ref, degb_ref,
              w2_ref, b1_ref, g2_ref):
  dis = _dis(dega_ref[...], degb_ref[...])[:, None]
  z0 = jnp.maximum(dis * (a1a_ref[...] + gc_ref[0]) + b1_ref[0, :H1], 0.0)
  z1 = jnp.maximum(dis * (a1b_ref[...] + gc_ref[1]) + b1_ref[0, H1:], 0.0)
  z = jnp.concatenate([z0, z1], axis=1)
  g2_ref[...] = jnp.dot(z, w2_ref[...], preferred_element_type=jnp.float32) * dis


def _mm2(a1a, a1b, gc, dega, degb, w2, b1):
  return pl.pallas_call(
      _mm2_body,
      grid=(N // RB,),
      in_specs=[
          pl.BlockSpec((RB, H1), lambda i: (i, 0)),
          pl.BlockSpec((RB, H1), lambda i: (i, 0)),
          pl.BlockSpec((2, RB, H1), lambda i: (0, i, 0)),
          pl.BlockSpec((RB, DEGW), lambda i: (i, 0)),
          pl.BlockSpec((RB, DEGW), lambda i: (i, 0)),
          pl.BlockSpec((D_HID, D_OUT), lambda i: (0, 0)),
          pl.BlockSpec((1, D_HID), lambda i: (0, 0)),
      ],
      out_specs=pl.BlockSpec((RB, D_OUT), lambda i: (i, 0)),
      out_shape=jax.ShapeDtypeStruct((N, D_OUT), jnp.float32),
  )(a1a, a1b, gc, dega, degb, w2, b1)


def _fin_body(a2a_ref, a2b_ref, g2_ref, dega_ref, degb_ref,
              b2_ref, out_ref):
  dis = _dis(dega_ref[...], degb_ref[...])[:, None]
  out_ref[...] = (dis * (a2a_ref[...] + a2b_ref[...] + g2_ref[...])
                  + b2_ref[0, :])


def _fin(a2a, a2b, g2, dega, degb, b2):
  return pl.pallas_call(
      _fin_body,
      grid=(N // RB,),
      in_specs=[
          pl.BlockSpec((RB, D_OUT), lambda i: (i, 0)),
          pl.BlockSpec((RB, D_OUT), lambda i: (i, 0)),
          pl.BlockSpec((RB, D_OUT), lambda i: (i, 0)),
          pl.BlockSpec((RB, DEGW), lambda i: (i, 0)),
          pl.BlockSpec((RB, DEGW), lambda i: (i, 0)),
          pl.BlockSpec((1, D_OUT), lambda i: (0, 0)),
      ],
      out_specs=pl.BlockSpec((RB, D_OUT), lambda i: (i, 0)),
      out_shape=jax.ShapeDtypeStruct((N, D_OUT), jnp.float32),
  )(a2a, a2b, g2, dega, degb, b2)


def kernel(x, edge_index, W1, b1, W2, b2):
  e = edge_index.shape[1]
  gran = NC * NS * CHUNK * GRP
  epad = ((e + gran - 1) // gran) * gran
  src = jnp.concatenate(
      [edge_index[0], jnp.zeros((epad - e,), jnp.int32)])
  dst = jnp.concatenate(
      [edge_index[1], jnp.full((epad - e,), N, jnp.int32)])
  srcq = jnp.concatenate([src, src + N])

  ones = jnp.ones((CHUNK, W128), jnp.float32)
  zeros128 = jnp.zeros((RPT, W128), jnp.float32)
  zeros_h1 = jnp.zeros((RPT, H1), jnp.float32)
  zeros_h2 = jnp.zeros((RPT, D_OUT), jnp.float32)

  deg2 = _deg_kernel(dst, ones, zeros128)
  dega, degb = deg2[0, :N, :DEGW], deg2[1, :N, :DEGW]
  gc = _mm1(x, W1, dega, degb)
  gcat = gc.reshape(2 * N, H1)
  a1 = _agg_kernel(gcat, srcq, dst, zeros_h1)
  g2 = _mm2(a1[0, :N], a1[1, :N], gc, dega, degb, W2, b1.reshape(1, D_HID))
  a2 = _agg2_kernel(g2, src, dst, zeros_h2)
  return _fin(a2[0, :N], a2[1, :N], g2, dega, degb, b2.reshape(1, D_OUT))


# speedup vs baseline: 7.8408x; 1.0310x over previous
import functools

import jax
import jax.numpy as jnp
from jax import lax
from jax.experimental import pallas as pl
from jax.experimental.pallas import tpu as pltpu
from jax.experimental.pallas import tpu_sc as plsc

N = 10000
NPAD = 10240
NC, NS = 2, 16
RPT = NPAD // NS
CHUNK = 128
GRP = 4
AGRP = 2
D_IN, D_HID, D_OUT = 256, 256, 128
H1 = D_HID // NC
W128 = 128
DEGW = 16
RB = 1000

_mesh = functools.partial(
    plsc.VectorSubcoreMesh, core_axis_name="c", subcore_axis_name="s",
    num_cores=NC, num_subcores=NS)


def _deg_body(ngrp, dst_hbm, ones_hbm, zeros_hbm, deg_out,
              acc, idx0, idx1, idx2, idx3, ones_v, isem, ssem):
  c = lax.axis_index("c")
  s = lax.axis_index("s")
  idxd = [idx0, idx1, idx2, idx3]
  pltpu.sync_copy(zeros_hbm, acc.at[pl.ds(s * RPT, RPT)])
  pltpu.sync_copy(ones_hbm, ones_v)
  plsc.subcore_barrier()
  base = (c * NS + s) * ngrp * GRP

  def step(k, carry):
    q = (base + k * GRP) * CHUNK
    ld = [pltpu.async_copy(dst_hbm.at[pl.ds(q + j * CHUNK, CHUNK)],
                           idxd[j], isem) for j in range(GRP)]
    st = []
    for j in range(GRP):
      ld[j].wait()
      st.append(pltpu.async_copy(ones_v, acc.at[idxd[j]], ssem, add=True))
    for d in st:
      d.wait()
    return carry

  lax.fori_loop(0, ngrp, step, 0)
  plsc.subcore_barrier()
  pltpu.sync_copy(acc.at[pl.ds(s * RPT, RPT)],
                  deg_out.at[c, pl.ds(s * RPT, RPT)])


def _deg_kernel(dst_pad, ones, zeros):
  ngrp = dst_pad.shape[0] // (NC * NS * CHUNK * GRP)
  return pl.kernel(
      functools.partial(_deg_body, ngrp),
      out_type=jax.ShapeDtypeStruct((NC, NPAD, W128), jnp.float32),
      mesh=_mesh(),
      scratch_types=[
          pltpu.VMEM_SHARED((NPAD, W128), jnp.float32),
          pltpu.VMEM((CHUNK,), jnp.int32),
          pltpu.VMEM((CHUNK,), jnp.int32),
          pltpu.VMEM((CHUNK,), jnp.int32),
          pltpu.VMEM((CHUNK,), jnp.int32),
          pltpu.VMEM((CHUNK, W128), jnp.float32),
          pltpu.SemaphoreType.DMA,
          pltpu.SemaphoreType.DMA,
      ],
  )(dst_pad, ones, zeros)


def _agg_pipeline(ngrp, base, src_off, g_hbm, src_hbm, dst_hbm,
                  acc, idxs, idxd, rows, isem, gsem, ssem):
  grp = len(rows)

  def scatter_wait(j):
    pltpu.make_async_copy(rows[j], acc.at[idxd[j]], ssem).wait()

  def step(k, carry):
    q = (base + k * grp) * CHUNK
    ld = []
    for j in range(grp):
      @pl.when(k > 0)
      def _(j=j):
        scatter_wait(j)
      ld.append(pltpu.async_copy(
          src_hbm.at[pl.ds(src_off + q + j * CHUNK, CHUNK)], idxs[j], isem))
      ld.append(pltpu.async_copy(
          dst_hbm.at[pl.ds(q + j * CHUNK, CHUNK)], idxd[j], isem))
    gt = []
    for j in range(grp):
      ld[2 * j].wait()
      ld[2 * j + 1].wait()
      gt.append(pltpu.async_copy(g_hbm.at[idxs[j]], rows[j], gsem))
    for j in range(grp):
      gt[j].wait()
      pltpu.async_copy(rows[j], acc.at[idxd[j]], ssem, add=True)
    return carry

  lax.fori_loop(0, ngrp, step, 0)
  for j in range(grp):
    scatter_wait(j)


def _agg_body(ngrp, epad, gcat_hbm, srcq_hbm, dst_hbm, zeros_hbm,
              out, acc, is0, is1, id0, id1, r0, r1, isem, gsem, ssem):
  c = lax.axis_index("c")
  s = lax.axis_index("s")
  pltpu.sync_copy(zeros_hbm, acc.at[pl.ds(s * RPT, RPT)])
  plsc.subcore_barrier()
  _agg_pipeline(ngrp, s * ngrp * AGRP, c * epad, gcat_hbm, srcq_hbm, dst_hbm,
                acc, [is0, is1], [id0, id1], [r0, r1], isem, gsem, ssem)
  plsc.subcore_barrier()
  pltpu.sync_copy(acc.at[pl.ds(s * RPT, RPT)],
                  out.at[c, pl.ds(s * RPT, RPT)])


def _agg_scratch(hw):
  return ([pltpu.VMEM_SHARED((NPAD, hw), jnp.float32)]
          + [pltpu.VMEM((CHUNK,), jnp.int32) for _ in range(2 * AGRP)]
          + [pltpu.VMEM((CHUNK, hw), jnp.float32) for _ in range(AGRP)]
          + [pltpu.SemaphoreType.DMA] * 3)


def _agg_kernel(gcat, srcq, dst_pad, zeros):
  epad = dst_pad.shape[0]
  ngrp = epad // (NS * CHUNK * AGRP)
  return pl.kernel(
      functools.partial(_agg_body, ngrp, epad),
      out_type=jax.ShapeDtypeStruct((NC, NPAD, H1), jnp.float32),
      mesh=_mesh(),
      scratch_types=_agg_scratch(H1),
  )(gcat, srcq, dst_pad, zeros)


def _agg2_body(ngrp, g_hbm, src_hbm, dst_hbm, zeros_hbm,
               out, acc, is0, is1, id0, id1, r0, r1, isem, gsem, ssem):
  c = lax.axis_index("c")
  s = lax.axis_index("s")
  pltpu.sync_copy(zeros_hbm, acc.at[pl.ds(s * RPT, RPT)])
  plsc.subcore_barrier()
  _agg_pipeline(ngrp, (c * NS + s) * ngrp * AGRP, 0, g_hbm, src_hbm, dst_hbm,
                acc, [is0, is1], [id0, id1], [r0, r1], isem, gsem, ssem)
  plsc.subcore_barrier()
  pltpu.sync_copy(acc.at[pl.ds(s * RPT, RPT)],
                  out.at[c, pl.ds(s * RPT, RPT)])


def _agg2_kernel(g, src_pad, dst_pad, zeros):
  ngrp = src_pad.shape[0] // (NC * NS * CHUNK * AGRP)
  return pl.kernel(
      functools.partial(_agg2_body, ngrp),
      out_type=jax.ShapeDtypeStruct((NC, NPAD, D_OUT), jnp.float32),
      mesh=_mesh(),
      scratch_types=_agg_scratch(D_OUT),
  )(g, src_pad, dst_pad, zeros)


def _dis(dega, degb):
  deg = dega[:, 0] + degb[:, 0] + 1.0
  return lax.rsqrt(deg)


def _mm1_body(x_ref, w_ref, dega_ref, degb_ref, gc_ref):
  h = jnp.dot(x_ref[...], w_ref[...], preferred_element_type=jnp.float32)
  g = h * _dis(dega_ref[...], degb_ref[...])[:, None]
  gc_ref[0] = g[:, :H1]
  gc_ref[1] = g[:, H1:]


def _mm1(x, w1, dega, degb):
  return pl.pallas_call(
      _mm1_body,
      grid=(N // RB,),
      in_specs=[
          pl.BlockSpec((RB, D_IN), lambda i: (i, 0)),
          pl.BlockSpec((D_IN, D_HID), lambda i: (0, 0)),
          pl.BlockSpec((RB, DEGW), lambda i: (i, 0)),
          pl.BlockSpec((RB, DEGW), lambda i: (i, 0)),
      ],
      out_specs=pl.BlockSpec((2, RB, H1), lambda i: (0, i, 0)),
      out_shape=jax.ShapeDtypeStruct((2, N, H1), jnp.float32),
  )(x, w1, dega, degb)


def _mm2_body(a1a_ref, a1b_ref, gc_ref, dega_ref, degb_ref,
              w2_ref, b1_ref, g2_ref):
  dis = _dis(dega_ref[...], degb_ref[...])[:, None]
  z0 = jnp.maximum(dis * (a1a_ref[...] + gc_ref[0]) + b1_ref[0, :H1], 0.0)
  z1 = jnp.maximum(dis * (a1b_ref[...] + gc_ref[1]) + b1_ref[0, H1:], 0.0)
  z = jnp.concatenate([z0, z1], axis=1)
  g2_ref[...] = jnp.dot(z, w2_ref[...], preferred_element_type=jnp.float32) * dis


def _mm2(a1a, a1b, gc, dega, degb, w2, b1):
  return pl.pallas_call(
      _mm2_body,
      grid=(N // RB,),
      in_specs=[
          pl.BlockSpec((RB, H1), lambda i: (i, 0)),
          pl.BlockSpec((RB, H1), lambda i: (i, 0)),
          pl.BlockSpec((2, RB, H1), lambda i: (0, i, 0)),
          pl.BlockSpec((RB, DEGW), lambda i: (i, 0)),
          pl.BlockSpec((RB, DEGW), lambda i: (i, 0)),
          pl.BlockSpec((D_HID, D_OUT), lambda i: (0, 0)),
          pl.BlockSpec((1, D_HID), lambda i: (0, 0)),
      ],
      out_specs=pl.BlockSpec((RB, D_OUT), lambda i: (i, 0)),
      out_shape=jax.ShapeDtypeStruct((N, D_OUT), jnp.float32),
  )(a1a, a1b, gc, dega, degb, w2, b1)


def _fin_body(a2a_ref, a2b_ref, g2_ref, dega_ref, degb_ref,
              b2_ref, out_ref):
  dis = _dis(dega_ref[...], degb_ref[...])[:, None]
  out_ref[...] = (dis * (a2a_ref[...] + a2b_ref[...] + g2_ref[...])
                  + b2_ref[0, :])


def _fin(a2a, a2b, g2, dega, degb, b2):
  return pl.pallas_call(
      _fin_body,
      grid=(N // RB,),
      in_specs=[
          pl.BlockSpec((RB, D_OUT), lambda i: (i, 0)),
          pl.BlockSpec((RB, D_OUT), lambda i: (i, 0)),
          pl.BlockSpec((RB, D_OUT), lambda i: (i, 0)),
          pl.BlockSpec((RB, DEGW), lambda i: (i, 0)),
          pl.BlockSpec((RB, DEGW), lambda i: (i, 0)),
          pl.BlockSpec((1, D_OUT), lambda i: (0, 0)),
      ],
      out_specs=pl.BlockSpec((RB, D_OUT), lambda i: (i, 0)),
      out_shape=jax.ShapeDtypeStruct((N, D_OUT), jnp.float32),
  )(a2a, a2b, g2, dega, degb, b2)


def kernel(x, edge_index, W1, b1, W2, b2):
  e = edge_index.shape[1]
  gran = NC * NS * CHUNK * GRP
  epad = ((e + gran - 1) // gran) * gran
  src = jnp.concatenate(
      [edge_index[0], jnp.zeros((epad - e,), jnp.int32)])
  dst = jnp.concatenate(
      [edge_index[1], jnp.full((epad - e,), N, jnp.int32)])
  srcq = jnp.concatenate([src, src + N])

  ones = jnp.ones((CHUNK, W128), jnp.float32)
  zeros128 = jnp.zeros((RPT, W128), jnp.float32)
  zeros_h1 = jnp.zeros((RPT, H1), jnp.float32)
  zeros_h2 = jnp.zeros((RPT, D_OUT), jnp.float32)

  deg2 = _deg_kernel(dst, ones, zeros128)
  dega, degb = deg2[0, :N, :DEGW], deg2[1, :N, :DEGW]
  gc = _mm1(x, W1, dega, degb)
  gcat = gc.reshape(2 * N, H1)
  a1 = _agg_kernel(gcat, srcq, dst, zeros_h1)
  g2 = _mm2(a1[0, :N], a1[1, :N], gc, dega, degb, W2, b1.reshape(1, D_HID))
  a2 = _agg2_kernel(g2, src, dst, zeros_h2)
  return _fin(a2[0, :N], a2[1, :N], g2, dega, degb, b2.reshape(1, D_OUT))
